# baseline (speedup 1.0000x reference)
import functools

import jax
import jax.numpy as jnp
from jax.experimental import pallas as pl
from jax.experimental.pallas import tpu as pltpu


def _silu(u):
    return u * jax.nn.sigmoid(u)


def _softplus(v):
    return jnp.maximum(v, 0.0) + jnp.log(1.0 + jnp.exp(-jnp.abs(v)))


def _expert_kernel(x_ref, w_in_ref, conv_w_ref, conv_b_ref, dt_bias_ref,
                   a_log_ref, d_par_ref, norm_w_ref, w_out_ref, out_ref,
                   h_ref, tail_ref, *, Q, n_heads, headdim, d_state, d_inner,
                   conv_k):
    c = pl.program_id(1)
    conv_dim = d_inner + 2 * d_state

    @pl.when(c == 0)
    def _():
        h_ref[...] = jnp.zeros_like(h_ref)
        tail_ref[...] = jnp.zeros_like(tail_ref)

    bf = jnp.bfloat16
    xb = x_ref[0]
    zxbcdt = jnp.dot(xb.astype(bf), w_in_ref[0],
                     preferred_element_type=jnp.float32)
    z = zxbcdt[:, :d_inner]
    xbc_raw = zxbcdt[:, d_inner:d_inner + conv_dim]
    dt_raw = zxbcdt[:, d_inner + conv_dim:]

    padded = jnp.concatenate([tail_ref[...], xbc_raw], axis=0)
    conv = jnp.broadcast_to(conv_b_ref[0], (Q, conv_dim))
    off = 8 - (conv_k - 1)
    for k in range(conv_k):
        conv = conv + padded[off + k:off + k + Q, :] * conv_w_ref[0, k:k + 1, :]
    tail_ref[...] = xbc_raw[Q - 8:, :]
    xbc = _silu(conv).astype(bf)

    bt_bf = xbc[:, d_inner:d_inner + d_state]
    ct_bf = xbc[:, d_inner + d_state:]
    xs_all = xbc[:, :d_inner]

    g = jax.lax.dot_general(ct_bf, bt_bf, (((1,), (1,)), ((), ())),
                            preferred_element_type=jnp.float32)
    rows = jax.lax.broadcasted_iota(jnp.int32, (Q, Q), 0)
    cols = jax.lax.broadcasted_iota(jnp.int32, (Q, Q), 1)
    causal = rows >= cols

    dt_t = _softplus(jnp.transpose(dt_raw) + dt_bias_ref[0])
    a_neg = -jnp.exp(a_log_ref[0])
    aa_t = dt_t * a_neg
    triu = jnp.where(rows <= cols, 1.0, 0.0)
    cum_t = jax.lax.dot_general(aa_t, triu, (((1,), (0,)), ((), ())),
                                preferred_element_type=jnp.float32)
    cum = jnp.transpose(cum_t)
    exp_cum_t = jnp.exp(cum_t)
    last_col = cum_t[:, Q - 1:Q]
    exp_tot_t = jnp.exp(last_col)
    exp_l_t = jnp.exp(last_col - cum_t)

    jcol = jax.lax.broadcasted_iota(jnp.int32, (n_heads, d_inner), 1)
    hrow = jax.lax.broadcasted_iota(jnp.int32, (n_heads, d_inner), 0)
    rep_bf = jnp.where(jcol // headdim == hrow, 1.0, 0.0).astype(bf)
    rep_f32 = rep_bf.astype(jnp.float32)
    tdims = (((0,), (0,)), ((), ()))
    dt_rep = jax.lax.dot_general(
        dt_t.astype(bf), rep_bf, tdims,
        preferred_element_type=jnp.float32).astype(bf)
    exp_cum_rep = jax.lax.dot_general(
        exp_cum_t.astype(bf), rep_bf, tdims,
        preferred_element_type=jnp.float32).astype(bf)
    exp_l_rep = jax.lax.dot_general(
        exp_l_t.astype(bf), rep_bf, tdims,
        preferred_element_type=jnp.float32).astype(bf)
    exp_tot_rep = jax.lax.dot_general(exp_tot_t, rep_f32, tdims,
                                      preferred_element_type=jnp.float32)

    dtx_all = dt_rep * xs_all
    h2 = h_ref[...]
    y_inter = exp_cum_rep * jax.lax.dot_general(
        ct_bf, h2.astype(bf), (((1,), (0,)), ((), ())),
        preferred_element_type=jnp.float32).astype(bf)
    h_ref[...] = exp_tot_rep * h2 + jax.lax.dot_general(
        bt_bf, exp_l_rep * dtx_all,
        (((0,), (0,)), ((), ())), preferred_element_type=jnp.float32)

    ys = []
    for h in range(n_heads):
        cum_h = cum[:, h:h + 1]
        diff = jnp.where(causal, cum_h - cum_t[h:h + 1, :], -jnp.inf)
        s = (g * jnp.exp(diff)).astype(bf)
        y = jax.lax.dot_general(s, dtx_all[:, h * headdim:(h + 1) * headdim],
                                (((1,), (0,)), ((), ())),
                                preferred_element_type=jnp.float32)
        ys.append(y)

    yv = jnp.concatenate(ys, axis=1)
    yv = yv + (y_inter + d_par_ref[0] * xs_all).astype(jnp.float32)
    yv = yv * _silu(z)
    yv = yv * jax.lax.rsqrt(jnp.mean(yv * yv, axis=1, keepdims=True) + 1e-5)
    yv = yv * norm_w_ref[0]
    out_ref[0] = jnp.dot(yv.astype(jnp.bfloat16), w_out_ref[0],
                         preferred_element_type=jnp.float32)


def _combine_kernel(x_ref, rw_ref, eo_ref, out_ref, *, n_exp):
    xb = x_ref[0]
    logits = jax.lax.dot_general(xb, rw_ref[...], (((1,), (1,)), ((), ())),
                                 preferred_element_type=jnp.float32)
    w = jax.nn.softmax(logits, axis=-1)
    acc = w[:, 0:1] * eo_ref[0]
    for e in range(1, n_exp):
        acc = acc + w[:, e:e + 1] * eo_ref[e]
    out_ref[0] = acc


def kernel(x, router_w, W_in, conv_w, conv_b, dt_bias, A_log, D_param,
           norm_w, W_out):
    B, T, D = x.shape
    E, d_inner, _ = W_out.shape
    n_heads = dt_bias.shape[1]
    headdim = d_inner // n_heads
    conv_dim, conv_k = conv_w.shape[1], conv_w.shape[2]
    d_state = (conv_dim - d_inner) // 2
    d_in_proj = W_in.shape[2]
    Q = min(128, T)

    w_in_bf = W_in.astype(jnp.bfloat16)
    w_out_bf = W_out.astype(jnp.bfloat16)
    conv_w_t = jnp.transpose(conv_w, (0, 2, 1))
    dt_bias3 = dt_bias[:, :, None]
    a_log3 = A_log[:, :, None]
    d_par3 = jnp.repeat(D_param, headdim, axis=1)[:, None, :].astype(
        jnp.bfloat16)
    conv_b3 = conv_b[:, None, :]
    norm_w3 = norm_w[:, None, :]

    grid = (E, T // Q)
    eo = pl.pallas_call(
        functools.partial(_expert_kernel, Q=Q, n_heads=n_heads,
                          headdim=headdim, d_state=d_state, d_inner=d_inner,
                          conv_k=conv_k),
        grid=grid,
        in_specs=[
            pl.BlockSpec((1, Q, D), lambda e, c: (0, c, 0)),
            pl.BlockSpec((1, D, d_in_proj), lambda e, c: (e, 0, 0)),
            pl.BlockSpec((1, conv_k, conv_dim), lambda e, c: (e, 0, 0)),
            pl.BlockSpec((1, 1, conv_dim), lambda e, c: (e, 0, 0)),
            pl.BlockSpec((1, n_heads, 1), lambda e, c: (e, 0, 0)),
            pl.BlockSpec((1, n_heads, 1), lambda e, c: (e, 0, 0)),
            pl.BlockSpec((1, 1, d_inner), lambda e, c: (e, 0, 0)),
            pl.BlockSpec((1, 1, d_inner), lambda e, c: (e, 0, 0)),
            pl.BlockSpec((1, d_inner, D), lambda e, c: (e, 0, 0)),
        ],
        out_specs=pl.BlockSpec((1, Q, D), lambda e, c: (e, c, 0)),
        out_shape=jax.ShapeDtypeStruct((E, T, D), jnp.float32),
        scratch_shapes=[
            pltpu.VMEM((d_state, d_inner), jnp.float32),
            pltpu.VMEM((8, conv_dim), jnp.float32),
        ],
    )(x, w_in_bf, conv_w_t, conv_b3, dt_bias3, a_log3, d_par3, norm_w3,
      w_out_bf)

    Qc = min(256, T)
    out = pl.pallas_call(
        functools.partial(_combine_kernel, n_exp=E),
        grid=(T // Qc,),
        in_specs=[
            pl.BlockSpec((1, Qc, D), lambda c: (0, c, 0)),
            pl.BlockSpec((E, D), lambda c: (0, 0)),
            pl.BlockSpec((E, Qc, D), lambda c: (0, c, 0)),
        ],
        out_specs=pl.BlockSpec((1, Qc, D), lambda c: (0, c, 0)),
        out_shape=jax.ShapeDtypeStruct((B, T, D), jnp.float32),
    )(x, router_w, eo)
    return out


# speedup vs baseline: 1.0451x; 1.0451x over previous
import functools

import jax
import jax.numpy as jnp
from jax.experimental import pallas as pl
from jax.experimental.pallas import tpu as pltpu


def _silu(u):
    return u * jax.nn.sigmoid(u)


def _softplus(v):
    return jnp.maximum(v, 0.0) + jnp.log(1.0 + jnp.exp(-jnp.abs(v)))


def _expert_kernel(x_ref, w_in_ref, conv_w_ref, conv_b_ref, dt_bias_ref,
                   a_log_ref, d_par_ref, norm_w_ref, w_out_ref, out_ref,
                   h_ref, tail_ref, *, Q, n_heads, headdim, d_state, d_inner,
                   conv_k):
    c = pl.program_id(1)
    conv_dim = d_inner + 2 * d_state

    @pl.when(c == 0)
    def _():
        h_ref[...] = jnp.zeros_like(h_ref)
        tail_ref[...] = jnp.zeros_like(tail_ref)

    bf = jnp.bfloat16
    xb = x_ref[0]
    zxbcdt = jnp.dot(xb.astype(bf), w_in_ref[0],
                     preferred_element_type=jnp.float32)
    z = zxbcdt[:, :d_inner]
    xbc_raw = zxbcdt[:, d_inner:d_inner + conv_dim]
    dt_raw = zxbcdt[:, d_inner + conv_dim:]

    padded = jnp.concatenate([tail_ref[...], xbc_raw], axis=0)
    conv = jnp.broadcast_to(conv_b_ref[0], (Q, conv_dim))
    off = 8 - (conv_k - 1)
    for k in range(conv_k):
        conv = conv + padded[off + k:off + k + Q, :] * conv_w_ref[0, k:k + 1, :]
    tail_ref[...] = xbc_raw[Q - 8:, :]
    xbc = _silu(conv).astype(bf)

    bt_bf = xbc[:, d_inner:d_inner + d_state]
    ct_bf = xbc[:, d_inner + d_state:]
    xs_all = xbc[:, :d_inner]

    g = jax.lax.dot_general(ct_bf, bt_bf, (((1,), (1,)), ((), ())),
                            preferred_element_type=jnp.float32)
    rows = jax.lax.broadcasted_iota(jnp.int32, (Q, Q), 0)
    cols = jax.lax.broadcasted_iota(jnp.int32, (Q, Q), 1)
    causal = rows >= cols

    dt_t = _softplus(jnp.transpose(dt_raw) + dt_bias_ref[0])
    a_neg = -jnp.exp(a_log_ref[0])
    aa_t = dt_t * a_neg
    triu = jnp.where(rows <= cols, 1.0, 0.0)
    cum_t = jax.lax.dot_general(aa_t, triu, (((1,), (0,)), ((), ())),
                                preferred_element_type=jnp.float32)
    cum = jnp.transpose(cum_t)
    exp_cum_t = jnp.exp(cum_t)
    last_col = cum_t[:, Q - 1:Q]
    exp_tot_t = jnp.exp(last_col)
    exp_l_t = jnp.exp(last_col - cum_t)

    jcol = jax.lax.broadcasted_iota(jnp.int32, (n_heads, d_inner), 1)
    hrow = jax.lax.broadcasted_iota(jnp.int32, (n_heads, d_inner), 0)
    rep_bf = jnp.where(jcol // headdim == hrow, 1.0, 0.0).astype(bf)
    rep_f32 = rep_bf.astype(jnp.float32)
    tdims = (((0,), (0,)), ((), ()))
    dt_rep = jax.lax.dot_general(
        dt_t.astype(bf), rep_bf, tdims,
        preferred_element_type=jnp.float32).astype(bf)
    exp_cum_rep = jax.lax.dot_general(
        exp_cum_t.astype(bf), rep_bf, tdims,
        preferred_element_type=jnp.float32)
    exp_l_rep = jax.lax.dot_general(
        exp_l_t.astype(bf), rep_bf, tdims,
        preferred_element_type=jnp.float32).astype(bf)
    exp_tot_rep = jax.lax.dot_general(exp_tot_t, rep_f32, tdims,
                                      preferred_element_type=jnp.float32)

    dtx_all = dt_rep * xs_all
    h2 = h_ref[...]
    y_inter = exp_cum_rep * jax.lax.dot_general(
        ct_bf, h2.astype(bf), (((1,), (0,)), ((), ())),
        preferred_element_type=jnp.float32)
    h_ref[...] = exp_tot_rep * h2 + jax.lax.dot_general(
        bt_bf, exp_l_rep * dtx_all,
        (((0,), (0,)), ((), ())), preferred_element_type=jnp.float32)

    ys = []
    for h in range(n_heads):
        cum_h = cum[:, h:h + 1]
        diff = jnp.where(causal, cum_h - cum_t[h:h + 1, :], -jnp.inf)
        s = (g * jnp.exp(diff)).astype(bf)
        y = jax.lax.dot_general(s, dtx_all[:, h * headdim:(h + 1) * headdim],
                                (((1,), (0,)), ((), ())),
                                preferred_element_type=jnp.float32)
        ys.append(y)

    yv = jnp.concatenate(ys, axis=1) + y_inter
    yv = yv + (d_par_ref[0] * xs_all).astype(jnp.float32)
    yv = yv * _silu(z)
    yv = yv * jax.lax.rsqrt(jnp.mean(yv * yv, axis=1, keepdims=True) + 1e-5)
    yv = yv * norm_w_ref[0]
    out_ref[0] = jnp.dot(yv.astype(jnp.bfloat16), w_out_ref[0],
                         preferred_element_type=jnp.float32)


def _combine_kernel(x_ref, rw_ref, eo_ref, out_ref, *, n_exp):
    xb = x_ref[0]
    logits = jax.lax.dot_general(xb, rw_ref[...], (((1,), (1,)), ((), ())),
                                 preferred_element_type=jnp.float32)
    w = jax.nn.softmax(logits, axis=-1)
    acc = w[:, 0:1] * eo_ref[0]
    for e in range(1, n_exp):
        acc = acc + w[:, e:e + 1] * eo_ref[e]
    out_ref[0] = acc


def kernel(x, router_w, W_in, conv_w, conv_b, dt_bias, A_log, D_param,
           norm_w, W_out):
    B, T, D = x.shape
    E, d_inner, _ = W_out.shape
    n_heads = dt_bias.shape[1]
    headdim = d_inner // n_heads
    conv_dim, conv_k = conv_w.shape[1], conv_w.shape[2]
    d_state = (conv_dim - d_inner) // 2
    d_in_proj = W_in.shape[2]
    Q = min(256, T)

    w_in_bf = W_in.astype(jnp.bfloat16)
    w_out_bf = W_out.astype(jnp.bfloat16)
    conv_w_t = jnp.transpose(conv_w, (0, 2, 1))
    dt_bias3 = dt_bias[:, :, None]
    a_log3 = A_log[:, :, None]
    d_par3 = jnp.repeat(D_param, headdim, axis=1)[:, None, :].astype(
        jnp.bfloat16)
    conv_b3 = conv_b[:, None, :]
    norm_w3 = norm_w[:, None, :]

    grid = (E, T // Q)
    eo = pl.pallas_call(
        functools.partial(_expert_kernel, Q=Q, n_heads=n_heads,
                          headdim=headdim, d_state=d_state, d_inner=d_inner,
                          conv_k=conv_k),
        grid=grid,
        in_specs=[
            pl.BlockSpec((1, Q, D), lambda e, c: (0, c, 0)),
            pl.BlockSpec((1, D, d_in_proj), lambda e, c: (e, 0, 0)),
            pl.BlockSpec((1, conv_k, conv_dim), lambda e, c: (e, 0, 0)),
            pl.BlockSpec((1, 1, conv_dim), lambda e, c: (e, 0, 0)),
            pl.BlockSpec((1, n_heads, 1), lambda e, c: (e, 0, 0)),
            pl.BlockSpec((1, n_heads, 1), lambda e, c: (e, 0, 0)),
            pl.BlockSpec((1, 1, d_inner), lambda e, c: (e, 0, 0)),
            pl.BlockSpec((1, 1, d_inner), lambda e, c: (e, 0, 0)),
            pl.BlockSpec((1, d_inner, D), lambda e, c: (e, 0, 0)),
        ],
        out_specs=pl.BlockSpec((1, Q, D), lambda e, c: (e, c, 0)),
        out_shape=jax.ShapeDtypeStruct((E, T, D), jnp.float32),
        scratch_shapes=[
            pltpu.VMEM((d_state, d_inner), jnp.float32),
            pltpu.VMEM((8, conv_dim), jnp.float32),
        ],
    )(x, w_in_bf, conv_w_t, conv_b3, dt_bias3, a_log3, d_par3, norm_w3,
      w_out_bf)

    Qc = min(256, T)
    out = pl.pallas_call(
        functools.partial(_combine_kernel, n_exp=E),
        grid=(T // Qc,),
        in_specs=[
            pl.BlockSpec((1, Qc, D), lambda c: (0, c, 0)),
            pl.BlockSpec((E, D), lambda c: (0, 0)),
            pl.BlockSpec((E, Qc, D), lambda c: (0, c, 0)),
        ],
        out_specs=pl.BlockSpec((1, Qc, D), lambda c: (0, c, 0)),
        out_shape=jax.ShapeDtypeStruct((B, T, D), jnp.float32),
    )(x, router_w, eo)
    return out


# speedup vs baseline: 1.1048x; 1.0571x over previous
import functools

import jax
import jax.numpy as jnp
from jax.experimental import pallas as pl
from jax.experimental.pallas import tpu as pltpu


def _silu(u):
    return u * jax.nn.sigmoid(u)


def _softplus(v):
    return jnp.maximum(v, 0.0) + jnp.log(1.0 + jnp.exp(-jnp.abs(v)))


def _expert_kernel(x_ref, w_in_ref, conv_w_ref, conv_b_ref, dt_bias_ref,
                   a_log_ref, d_par_ref, norm_w_ref, w_out_ref, out_ref,
                   h_ref, tail_ref, *, Q, n_heads, headdim, d_state, d_inner,
                   conv_k):
    c = pl.program_id(1)
    conv_dim = d_inner + 2 * d_state

    @pl.when(c == 0)
    def _():
        h_ref[...] = jnp.zeros_like(h_ref)
        tail_ref[...] = jnp.zeros_like(tail_ref)

    bf = jnp.bfloat16
    xb = x_ref[0]
    zxbcdt = jnp.dot(xb.astype(bf), w_in_ref[0],
                     preferred_element_type=jnp.float32)
    z = zxbcdt[:, :d_inner]
    xbc_raw = zxbcdt[:, d_inner:d_inner + conv_dim]
    dt_raw = zxbcdt[:, d_inner + conv_dim:]

    padded = jnp.concatenate([tail_ref[...], xbc_raw], axis=0)
    conv = jnp.broadcast_to(conv_b_ref[0], (Q, conv_dim))
    off = 8 - (conv_k - 1)
    for k in range(conv_k):
        conv = conv + padded[off + k:off + k + Q, :] * conv_w_ref[0, k:k + 1, :]
    tail_ref[...] = xbc_raw[Q - 8:, :]
    xbc = _silu(conv)

    bt = xbc[:, d_inner:d_inner + d_state]
    ct = xbc[:, d_inner + d_state:]
    dt = _softplus(dt_raw + dt_bias_ref[0])
    a_neg = -jnp.exp(a_log_ref[0])

    bt_bf = bt.astype(bf)
    ct_bf = ct.astype(bf)
    g = jax.lax.dot_general(ct_bf, bt_bf, (((1,), (1,)), ((), ())),
                            preferred_element_type=jnp.float32)
    rows = jax.lax.broadcasted_iota(jnp.int32, (Q, Q), 0)
    cols = jax.lax.broadcasted_iota(jnp.int32, (Q, Q), 1)
    causal = rows >= cols

    tri = jnp.where(causal, 1.0, 0.0)
    cum = jax.lax.dot_general(tri, dt * a_neg, (((1,), (0,)), ((), ())),
                              preferred_element_type=jnp.float32)
    cum_t = jnp.transpose(cum)
    exp_cum = jnp.exp(cum)
    last_row = cum[Q - 1:Q, :]
    exp_tot = jnp.exp(last_row)
    w_all = dt * jnp.exp(last_row - cum)

    jcol = jax.lax.broadcasted_iota(jnp.int32, (n_heads, d_inner), 1)
    hrow = jax.lax.broadcasted_iota(jnp.int32, (n_heads, d_inner), 0)
    rep = jnp.where(jcol // headdim == hrow, 1.0, 0.0)
    rdims = (((1,), (0,)), ((), ()))
    dt_rep = jax.lax.dot_general(dt, rep, rdims,
                                 preferred_element_type=jnp.float32)
    exp_cum_rep = jax.lax.dot_general(exp_cum, rep, rdims,
                                      preferred_element_type=jnp.float32)
    w_rep = jax.lax.dot_general(w_all, rep, rdims,
                                preferred_element_type=jnp.float32)
    exp_tot_rep = jax.lax.dot_general(exp_tot, rep, rdims,
                                      preferred_element_type=jnp.float32)

    xs_all = xbc[:, :d_inner]
    dtx_all = (dt_rep * xs_all).astype(bf)
    h2 = h_ref[...]
    y_inter = exp_cum_rep * jax.lax.dot_general(
        ct_bf, h2.astype(bf), rdims,
        preferred_element_type=jnp.float32)
    h_ref[...] = exp_tot_rep * h2 + jax.lax.dot_general(
        bt_bf, (w_rep * xs_all).astype(bf),
        (((0,), (0,)), ((), ())), preferred_element_type=jnp.float32)

    ys = []
    for h in range(n_heads):
        cum_h = cum[:, h:h + 1]
        diff = jnp.where(causal, cum_h - cum_t[h:h + 1, :], -jnp.inf)
        s = (g * jnp.exp(diff)).astype(bf)
        y = jax.lax.dot_general(s, dtx_all[:, h * headdim:(h + 1) * headdim],
                                (((1,), (0,)), ((), ())),
                                preferred_element_type=jnp.float32)
        ys.append(y)

    yv = jnp.concatenate(ys, axis=1) + y_inter
    yv = yv + d_par_ref[0] * xs_all
    yv = yv * _silu(z)
    yv = yv * jax.lax.rsqrt(jnp.mean(yv * yv, axis=1, keepdims=True) + 1e-5)
    yv = yv * norm_w_ref[0]
    out_ref[0] = jnp.dot(yv.astype(jnp.bfloat16), w_out_ref[0],
                         preferred_element_type=jnp.float32)


def _combine_kernel(x_ref, rw_ref, eo_ref, out_ref, *, n_exp):
    xb = x_ref[0]
    logits = jax.lax.dot_general(xb, rw_ref[...], (((1,), (1,)), ((), ())),
                                 preferred_element_type=jnp.float32)
    w = jax.nn.softmax(logits, axis=-1)
    acc = w[:, 0:1] * eo_ref[0]
    for e in range(1, n_exp):
        acc = acc + w[:, e:e + 1] * eo_ref[e]
    out_ref[0] = acc


def kernel(x, router_w, W_in, conv_w, conv_b, dt_bias, A_log, D_param,
           norm_w, W_out):
    B, T, D = x.shape
    E, d_inner, _ = W_out.shape
    n_heads = dt_bias.shape[1]
    headdim = d_inner // n_heads
    conv_dim, conv_k = conv_w.shape[1], conv_w.shape[2]
    d_state = (conv_dim - d_inner) // 2
    d_in_proj = W_in.shape[2]
    Q = min(256, T)

    w_in_bf = W_in.astype(jnp.bfloat16)
    w_out_bf = W_out.astype(jnp.bfloat16)
    conv_w_t = jnp.transpose(conv_w, (0, 2, 1))
    dt_bias3 = dt_bias[:, None, :]
    a_log3 = A_log[:, None, :]
    d_par3 = jnp.repeat(D_param, headdim, axis=1)[:, None, :]
    conv_b3 = conv_b[:, None, :]
    norm_w3 = norm_w[:, None, :]

    grid = (E, T // Q)
    eo = pl.pallas_call(
        functools.partial(_expert_kernel, Q=Q, n_heads=n_heads,
                          headdim=headdim, d_state=d_state, d_inner=d_inner,
                          conv_k=conv_k),
        grid=grid,
        in_specs=[
            pl.BlockSpec((1, Q, D), lambda e, c: (0, c, 0)),
            pl.BlockSpec((1, D, d_in_proj), lambda e, c: (e, 0, 0)),
            pl.BlockSpec((1, conv_k, conv_dim), lambda e, c: (e, 0, 0)),
            pl.BlockSpec((1, 1, conv_dim), lambda e, c: (e, 0, 0)),
            pl.BlockSpec((1, 1, n_heads), lambda e, c: (e, 0, 0)),
            pl.BlockSpec((1, 1, n_heads), lambda e, c: (e, 0, 0)),
            pl.BlockSpec((1, 1, d_inner), lambda e, c: (e, 0, 0)),
            pl.BlockSpec((1, 1, d_inner), lambda e, c: (e, 0, 0)),
            pl.BlockSpec((1, d_inner, D), lambda e, c: (e, 0, 0)),
        ],
        out_specs=pl.BlockSpec((1, Q, D), lambda e, c: (e, c, 0)),
        out_shape=jax.ShapeDtypeStruct((E, T, D), jnp.float32),
        scratch_shapes=[
            pltpu.VMEM((d_state, d_inner), jnp.float32),
            pltpu.VMEM((8, conv_dim), jnp.float32),
        ],
    )(x, w_in_bf, conv_w_t, conv_b3, dt_bias3, a_log3, d_par3, norm_w3,
      w_out_bf)

    Qc = min(256, T)
    out = pl.pallas_call(
        functools.partial(_combine_kernel, n_exp=E),
        grid=(T // Qc,),
        in_specs=[
            pl.BlockSpec((1, Qc, D), lambda c: (0, c, 0)),
            pl.BlockSpec((E, D), lambda c: (0, 0)),
            pl.BlockSpec((E, Qc, D), lambda c: (0, c, 0)),
        ],
        out_specs=pl.BlockSpec((1, Qc, D), lambda c: (0, c, 0)),
        out_shape=jax.ShapeDtypeStruct((B, T, D), jnp.float32),
    )(x, router_w, eo)
    return out


# speedup vs baseline: 1.1204x; 1.0142x over previous
import functools

import jax
import jax.numpy as jnp
from jax.experimental import pallas as pl
from jax.experimental.pallas import tpu as pltpu


def _silu(u):
    return u * jax.nn.sigmoid(u)


def _softplus(v):
    return jnp.maximum(v, 0.0) + jnp.log(1.0 + jnp.exp(-jnp.abs(v)))


def _expert_kernel(x_ref, w_in_ref, conv_w_ref, conv_b_ref, dt_bias_ref,
                   a_log_ref, d_par_ref, norm_w_ref, w_out_ref, out_ref,
                   h_ref, tail_ref, *, Q, n_heads, headdim, d_state, d_inner,
                   conv_k):
    c = pl.program_id(1)
    conv_dim = d_inner + 2 * d_state

    @pl.when(c == 0)
    def _():
        h_ref[...] = jnp.zeros_like(h_ref)
        tail_ref[...] = jnp.zeros_like(tail_ref)

    bf = jnp.bfloat16
    xb = x_ref[0]
    zxbcdt = jnp.dot(xb.astype(bf), w_in_ref[0],
                     preferred_element_type=jnp.float32)
    z = zxbcdt[:, :d_inner]
    xbc_raw = zxbcdt[:, d_inner:d_inner + conv_dim]
    dt_raw = zxbcdt[:, d_inner + conv_dim:]

    padded = jnp.concatenate([tail_ref[...], xbc_raw], axis=0)
    conv = jnp.broadcast_to(conv_b_ref[0], (Q, conv_dim))
    off = 8 - (conv_k - 1)
    for k in range(conv_k):
        conv = conv + padded[off + k:off + k + Q, :] * conv_w_ref[0, k:k + 1, :]
    tail_ref[...] = xbc_raw[Q - 8:, :]
    xbc = _silu(conv)

    bt = xbc[:, d_inner:d_inner + d_state]
    ct = xbc[:, d_inner + d_state:]
    dt = _softplus(dt_raw + dt_bias_ref[0])
    a_neg = -jnp.exp(a_log_ref[0])

    bt_bf = bt.astype(bf)
    ct_bf = ct.astype(bf)
    g = jax.lax.dot_general(ct_bf, bt_bf, (((1,), (1,)), ((), ())),
                            preferred_element_type=jnp.float32)
    rows = jax.lax.broadcasted_iota(jnp.int32, (Q, Q), 0)
    cols = jax.lax.broadcasted_iota(jnp.int32, (Q, Q), 1)
    causal = rows >= cols

    tri = jnp.where(causal, 1.0, 0.0)
    cum = jax.lax.dot_general(tri, dt * a_neg, (((1,), (0,)), ((), ())),
                              preferred_element_type=jnp.float32)
    cum_t = jnp.transpose(cum)
    exp_cum = jnp.exp(cum)
    last_row = cum[Q - 1:Q, :]
    exp_tot = jnp.exp(last_row)
    w_all = dt * jnp.exp(last_row - cum)

    jcol = jax.lax.broadcasted_iota(jnp.int32, (n_heads, d_inner), 1)
    hrow = jax.lax.broadcasted_iota(jnp.int32, (n_heads, d_inner), 0)
    rep = jnp.where(jcol // headdim == hrow, 1.0, 0.0)
    rdims = (((1,), (0,)), ((), ()))
    dt_rep = jax.lax.dot_general(dt, rep, rdims,
                                 preferred_element_type=jnp.float32)
    exp_cum_rep = jax.lax.dot_general(exp_cum, rep, rdims,
                                      preferred_element_type=jnp.float32)
    w_rep = jax.lax.dot_general(w_all, rep, rdims,
                                preferred_element_type=jnp.float32)
    exp_tot_rep = jax.lax.dot_general(exp_tot, rep, rdims,
                                      preferred_element_type=jnp.float32)

    xs_all = xbc[:, :d_inner]
    dtx_all = (dt_rep * xs_all).astype(bf)
    h2 = h_ref[...]
    y_inter = exp_cum_rep * jax.lax.dot_general(
        ct_bf, h2.astype(bf), rdims,
        preferred_element_type=jnp.float32)
    h_ref[...] = exp_tot_rep * h2 + jax.lax.dot_general(
        bt_bf, (w_rep * xs_all).astype(bf),
        (((0,), (0,)), ((), ())), preferred_element_type=jnp.float32)

    ys = []
    for h in range(n_heads):
        cum_h = cum[:, h:h + 1]
        diff = jnp.where(causal, cum_h - cum_t[h:h + 1, :], -jnp.inf)
        s = (g * jnp.exp(diff)).astype(bf)
        y = jax.lax.dot_general(s, dtx_all[:, h * headdim:(h + 1) * headdim],
                                (((1,), (0,)), ((), ())),
                                preferred_element_type=jnp.float32)
        ys.append(y)

    yv = jnp.concatenate(ys, axis=1) + y_inter
    yv = yv + d_par_ref[0] * xs_all
    yv = yv * _silu(z)
    yv = yv * jax.lax.rsqrt(jnp.mean(yv * yv, axis=1, keepdims=True) + 1e-5)
    yv = yv * norm_w_ref[0]
    out_ref[0] = jnp.dot(yv.astype(jnp.bfloat16), w_out_ref[0],
                         preferred_element_type=jnp.float32).astype(bf)


def _combine_kernel(x_ref, rw_ref, eo_ref, out_ref, *, n_exp):
    xb = x_ref[0]
    logits = jax.lax.dot_general(xb, rw_ref[...], (((1,), (1,)), ((), ())),
                                 preferred_element_type=jnp.float32)
    w = jax.nn.softmax(logits, axis=-1)
    acc = w[:, 0:1] * eo_ref[0]
    for e in range(1, n_exp):
        acc = acc + w[:, e:e + 1] * eo_ref[e]
    out_ref[0] = acc


def kernel(x, router_w, W_in, conv_w, conv_b, dt_bias, A_log, D_param,
           norm_w, W_out):
    B, T, D = x.shape
    E, d_inner, _ = W_out.shape
    n_heads = dt_bias.shape[1]
    headdim = d_inner // n_heads
    conv_dim, conv_k = conv_w.shape[1], conv_w.shape[2]
    d_state = (conv_dim - d_inner) // 2
    d_in_proj = W_in.shape[2]
    Q = min(256, T)

    w_in_bf = W_in.astype(jnp.bfloat16)
    w_out_bf = W_out.astype(jnp.bfloat16)
    conv_w_t = jnp.transpose(conv_w, (0, 2, 1))
    dt_bias3 = dt_bias[:, None, :]
    a_log3 = A_log[:, None, :]
    d_par3 = jnp.repeat(D_param, headdim, axis=1)[:, None, :]
    conv_b3 = conv_b[:, None, :]
    norm_w3 = norm_w[:, None, :]

    grid = (E, T // Q)
    eo = pl.pallas_call(
        functools.partial(_expert_kernel, Q=Q, n_heads=n_heads,
                          headdim=headdim, d_state=d_state, d_inner=d_inner,
                          conv_k=conv_k),
        grid=grid,
        in_specs=[
            pl.BlockSpec((1, Q, D), lambda e, c: (0, c, 0)),
            pl.BlockSpec((1, D, d_in_proj), lambda e, c: (e, 0, 0)),
            pl.BlockSpec((1, conv_k, conv_dim), lambda e, c: (e, 0, 0)),
            pl.BlockSpec((1, 1, conv_dim), lambda e, c: (e, 0, 0)),
            pl.BlockSpec((1, 1, n_heads), lambda e, c: (e, 0, 0)),
            pl.BlockSpec((1, 1, n_heads), lambda e, c: (e, 0, 0)),
            pl.BlockSpec((1, 1, d_inner), lambda e, c: (e, 0, 0)),
            pl.BlockSpec((1, 1, d_inner), lambda e, c: (e, 0, 0)),
            pl.BlockSpec((1, d_inner, D), lambda e, c: (e, 0, 0)),
        ],
        out_specs=pl.BlockSpec((1, Q, D), lambda e, c: (e, c, 0)),
        out_shape=jax.ShapeDtypeStruct((E, T, D), jnp.bfloat16),
        scratch_shapes=[
            pltpu.VMEM((d_state, d_inner), jnp.float32),
            pltpu.VMEM((8, conv_dim), jnp.float32),
        ],
    )(x, w_in_bf, conv_w_t, conv_b3, dt_bias3, a_log3, d_par3, norm_w3,
      w_out_bf)

    Qc = min(256, T)
    out = pl.pallas_call(
        functools.partial(_combine_kernel, n_exp=E),
        grid=(T // Qc,),
        in_specs=[
            pl.BlockSpec((1, Qc, D), lambda c: (0, c, 0)),
            pl.BlockSpec((E, D), lambda c: (0, 0)),
            pl.BlockSpec((E, Qc, D), lambda c: (0, c, 0)),
        ],
        out_specs=pl.BlockSpec((1, Qc, D), lambda c: (0, c, 0)),
        out_shape=jax.ShapeDtypeStruct((B, T, D), jnp.float32),
    )(x, router_w, eo)
    return out
